# Initial kernel scaffold; baseline (speedup 1.0000x reference)
import jax
import jax.numpy as jnp
from jax.experimental import pallas as pl


def kernel(x, W1, b1, W2, b2, fw1, fb1, fw2, fb2, fw3, fb3, edge_index, batch):
    raise NotImplementedError("write your pallas kernel here")


# speedup vs baseline: 32.7472x; 32.7472x over previous
import functools

import jax
import jax.numpy as jnp
from jax import lax
from jax.experimental import pallas as pl
from jax.experimental.pallas import tpu as pltpu
from jax.experimental.pallas import tpu_sc as plsc

N = 10000
E = 320000
D = 128
G = 64
F = 128
NC, NS, L = 2, 16, 16
NW = NC * NS
NP = 10240
CPS = NP // NS
EPW = E // NW
ECH = 80
NCH = EPW // ECH

_MESH = dict(core_axis_name="c", subcore_axis_name="s", num_cores=NC,
             num_subcores=NS)
_SC_PARAMS = pltpu.CompilerParams(needs_layout_passes=False)
_HI = jax.lax.Precision.HIGHEST


def _bf(a):
    return a.astype(jnp.bfloat16).astype(jnp.float32)


def _tc_h1_body(x_ref, w1_ref, h1_ref):
    xb = _bf(x_ref[...])
    wb = _bf(w1_ref[...])
    h = jax.lax.dot_general(xb, wb, (((1,), (0,)), ((), ())),
                            precision=_HI,
                            preferred_element_type=jnp.float32)
    h1_ref[...] = jnp.pad(h, ((0, NP - N), (0, F - 60)))


_tc_h1 = pl.pallas_call(
    _tc_h1_body,
    out_shape=jax.ShapeDtypeStruct((NP, F), jnp.float32),
)


def _k1_body(edge_hbm, deg2_hbm, dstv, acc, red0, red1, accsh):
    cid = lax.axis_index("c")
    sid = lax.axis_index("s")
    w = cid * NS + sid
    pltpu.sync_copy(edge_hbm.at[pl.ds(E + w * EPW, EPW)], dstv)
    z16 = jnp.zeros((L,), jnp.float32)
    ones = jnp.ones((L,), jnp.float32)

    def zero_body(i, _):
        acc[pl.ds(i * L, L)] = z16
        return 0
    lax.fori_loop(0, NP // L, zero_body, 0)

    def edge_body(i, _):
        idx = dstv[pl.ds(i * L, L)]
        plsc.addupdate_scatter(acc, [idx], ones)
        return 0
    lax.fori_loop(0, EPW // L, edge_body, 0)

    pltpu.sync_copy(acc, accsh.at[sid])
    plsc.subcore_barrier()
    base = sid * CPS

    def zred(i, _):
        red0[pl.ds(i * L, L)] = z16
        return 0
    lax.fori_loop(0, CPS // L, zred, 0)
    for t in range(NS):
        pltpu.sync_copy(accsh.at[t, pl.ds(base, CPS)], red1)
        def addb(i, _, _t=t):
            sl = pl.ds(i * L, L)
            red0[sl] = red0[sl] + red1[sl]
            return 0
        lax.fori_loop(0, CPS // L, addb, 0)
    pltpu.sync_copy(red0, deg2_hbm.at[pl.ds(cid * NP + base, CPS)])


_k1 = pl.kernel(
    _k1_body,
    out_type=jax.ShapeDtypeStruct((NC * NP,), jnp.float32),
    mesh=plsc.VectorSubcoreMesh(**_MESH),
    compiler_params=_SC_PARAMS,
    scratch_types=[
        pltpu.VMEM((EPW,), jnp.int32),
        pltpu.VMEM((NP,), jnp.float32),
        pltpu.VMEM((CPS,), jnp.float32),
        pltpu.VMEM((CPS,), jnp.float32),
        pltpu.VMEM_SHARED((NS, NP), jnp.float32),
    ],
)


def _tc_mid_body(deg2_ref, h1_ref, dis_ref, h1s_ref):
    deg = deg2_ref[0:1, :] + deg2_ref[1:2, :] + 1.0
    dis = jax.lax.rsqrt(deg)
    dis_ref[...] = dis
    h1s_ref[...] = h1_ref[...] * dis.reshape(NP, 1)


_tc_mid = pl.pallas_call(
    _tc_mid_body,
    out_shape=(jax.ShapeDtypeStruct((1, NP), jnp.float32),
               jax.ShapeDtypeStruct((NP, F), jnp.float32)),
)


def _edge_body(edge_hbm, h1s_hbm, hraw_hbm,
               srcv, dstv, rowbuf, zrows, sidx, didx, haccsp):
    cid = lax.axis_index("c")
    sid = lax.axis_index("s")
    w = cid * NS + sid
    base = sid * CPS
    pltpu.sync_copy(edge_hbm.at[pl.ds(w * EPW, EPW)], srcv)
    pltpu.sync_copy(edge_hbm.at[pl.ds(E + w * EPW, EPW)], dstv)
    z16 = jnp.zeros((L,), jnp.float32)
    def zr_body(i, _):
        for jj in range(F // L):
            zrows[i, pl.ds(jj * L, L)] = z16
        return 0
    lax.fori_loop(0, ECH, zr_body, 0)

    def zacc_body(i, _):
        pltpu.sync_copy(zrows, haccsp.at[pl.ds(base + i * ECH, ECH)])
        return 0
    lax.fori_loop(0, CPS // ECH, zacc_body, 0)
    plsc.subcore_barrier()

    def chunk_body(c, _):
        eb = c * ECH
        for j in range(ECH // L):
            sl = pl.ds(j * L, L)
            sidx[0, sl] = srcv[pl.ds(eb + j * L, L)]
            didx[0, sl] = dstv[pl.ds(eb + j * L, L)]
        pltpu.sync_copy(h1s_hbm.at[sidx.at[0]], rowbuf)
        pltpu.sync_copy(rowbuf, haccsp.at[didx.at[0]], add=True)
        return 0
    lax.fori_loop(0, NCH, chunk_body, 0)
    plsc.subcore_barrier()

    def out_body(i, _):
        pltpu.sync_copy(haccsp.at[pl.ds(base + i * ECH, ECH)], rowbuf)
        pltpu.sync_copy(rowbuf,
                        hraw_hbm.at[pl.ds(cid * NP + base + i * ECH, ECH)])
        return 0
    lax.fori_loop(0, CPS // ECH, out_body, 0)


_sc_edge = pl.kernel(
    _edge_body,
    out_type=jax.ShapeDtypeStruct((NC * NP, F), jnp.float32),
    mesh=plsc.VectorSubcoreMesh(**_MESH),
    compiler_params=_SC_PARAMS,
    scratch_types=[
        pltpu.VMEM((EPW,), jnp.int32),
        pltpu.VMEM((EPW,), jnp.int32),
        pltpu.VMEM((ECH, F), jnp.float32),
        pltpu.VMEM((ECH, F), jnp.float32),
        pltpu.VMEM((1, ECH), jnp.int32),
        pltpu.VMEM((1, ECH), jnp.int32),
        pltpu.VMEM_SHARED((NP, F), jnp.float32),
    ],
)


def _cmat_body(edge_hbm, batch_hbm, dis_hbm, c_hbm,
               srcv, dstv, disv, batchv, cval, zbuf, cidx, csp):
    cid = lax.axis_index("c")
    sid = lax.axis_index("s")
    w = cid * NS + sid
    CSL = G * NP // NS
    ZL = ECH * F
    pltpu.sync_copy(edge_hbm.at[pl.ds(w * EPW, EPW)], srcv)
    pltpu.sync_copy(edge_hbm.at[pl.ds(E + w * EPW, EPW)], dstv)
    pltpu.sync_copy(dis_hbm, disv)
    pltpu.sync_copy(batch_hbm, batchv)
    z16 = jnp.zeros((L,), jnp.float32)

    def zb_body(i, _):
        zbuf[pl.ds(i * L, L)] = z16
        return 0
    lax.fori_loop(0, ZL // L, zb_body, 0)

    def zc_body(i, _):
        pltpu.sync_copy(zbuf, csp.at[pl.ds(sid * CSL + i * ZL, ZL)])
        return 0
    lax.fori_loop(0, CSL // ZL, zc_body, 0)
    plsc.subcore_barrier()

    def chunk_body(c, _):
        eb = c * ECH
        for j in range(ECH // L):
            sl = pl.ds(j * L, L)
            s16 = srcv[pl.ds(eb + j * L, L)]
            d16 = dstv[pl.ds(eb + j * L, L)]
            b16 = plsc.load_gather(batchv, [d16])
            w16 = (plsc.load_gather(disv, [s16])
                   * plsc.load_gather(disv, [d16]))
            cidx[0, sl] = b16 * NP + s16
            cval[sl] = w16
        pltpu.sync_copy(cval, csp.at[cidx.at[0]], add=True)
        return 0
    lax.fori_loop(0, NCH, chunk_body, 0)
    plsc.subcore_barrier()

    def outc_body(i, _):
        off = sid * CSL + i * ZL
        pltpu.sync_copy(csp.at[pl.ds(off, ZL)], zbuf)
        pltpu.sync_copy(zbuf, c_hbm.at[pl.ds(cid * G * NP + off, ZL)])
        return 0
    lax.fori_loop(0, CSL // ZL, outc_body, 0)


_sc_cmat = pl.kernel(
    _cmat_body,
    out_type=jax.ShapeDtypeStruct((NC * G * NP,), jnp.float32),
    mesh=plsc.VectorSubcoreMesh(**_MESH),
    compiler_params=_SC_PARAMS,
    scratch_types=[
        pltpu.VMEM((EPW,), jnp.int32),
        pltpu.VMEM((EPW,), jnp.int32),
        pltpu.VMEM((NP,), jnp.float32),
        pltpu.VMEM((N,), jnp.int32),
        pltpu.VMEM((ECH,), jnp.float32),
        pltpu.VMEM((ECH * F,), jnp.float32),
        pltpu.VMEM((1, ECH), jnp.int32),
        pltpu.VMEM_SHARED((G * NP,), jnp.float32),
    ],
)


def _tc_fin_body(hraw_ref, h1s_ref, dis_ref, c2_ref, batch_ref, b1_ref,
                 w2_ref, b2_ref, fw1_ref, fb1_ref, fw2_ref, fb2_ref,
                 fw3_ref, fb3_ref, out_ref):
    dis = dis_ref[0, :]
    hsum = hraw_ref[0] + hraw_ref[1] + h1s_ref[...]
    lane = lax.broadcasted_iota(jnp.int32, (1, F), 1)
    b1pad = jnp.where(lane < 60, jnp.pad(b1_ref[...], (0, F - 60))[None, :],
                      0.0)
    h1 = hsum * dis[:, None] + b1pad
    bfh1 = _bf(h1)
    gi = lax.broadcasted_iota(jnp.int32, (G, NP), 0)
    ni = lax.broadcasted_iota(jnp.int32, (G, NP), 1)
    bpad = jnp.pad(batch_ref[...], (0, NP - N), constant_values=-1)
    onehot = jnp.where((gi == bpad[None, :]) & (ni < N), 1.0, 0.0)
    cmat = (c2_ref[0] + c2_ref[1] + onehot * (dis * dis)[None, :])
    gpool = jax.lax.dot_general(cmat, bfh1, (((1,), (0,)), ((), ())),
                                precision=_HI,
                                preferred_element_type=jnp.float32)
    w2pad = jnp.pad(_bf(w2_ref[...]), ((0, F - 60), (0, 0)))
    g50 = jax.lax.dot_general(gpool, w2pad, (((1,), (0,)), ((), ())),
                              precision=_HI,
                              preferred_element_type=jnp.float32)
    cnt = jnp.sum(onehot, axis=1, keepdims=True)
    g50 = g50 + cnt * b2_ref[...][None, :]
    m = jax.lax.dot_general(_bf(g50), _bf(fw1_ref[...]),
                            (((1,), (0,)), ((), ())), precision=_HI,
                            preferred_element_type=jnp.float32)
    m = m + fb1_ref[...][None, :]
    m = jax.lax.dot_general(_bf(m), _bf(fw2_ref[...]),
                            (((1,), (0,)), ((), ())), precision=_HI,
                            preferred_element_type=jnp.float32)
    m = m + fb2_ref[...][None, :]
    out = jnp.sum(_bf(m) * _bf(fw3_ref[...])[:, 0][None, :], axis=1,
                  keepdims=True)
    out_ref[...] = out + fb3_ref[...][None, :]


_tc_fin = pl.pallas_call(
    _tc_fin_body,
    out_shape=jax.ShapeDtypeStruct((G, 1), jnp.float32),
)


def kernel(x, W1, b1, W2, b2, fw1, fb1, fw2, fb2, fw3, fb3, edge_index,
           batch):
    eflat = edge_index.reshape(2 * E)
    h1m = _tc_h1(x, W1)
    deg2 = _k1(eflat)
    dis, h1s = _tc_mid(deg2.reshape(2, NP), h1m)
    hraw = _sc_edge(eflat, h1s)
    cflat = _sc_cmat(eflat, batch, dis.reshape(NP))
    return _tc_fin(hraw.reshape(NC, NP, F), h1s, dis,
                   cflat.reshape(NC, G, NP), batch,
                   b1, W2, b2, fw1, fb1, fw2, fb2, fw3, fb3)
